# baseline (speedup 1.0000x reference)
import functools

import jax
import jax.numpy as jnp
from jax import lax
from jax.experimental import pallas as pl
from jax.experimental.pallas import tpu as pltpu
from jax.experimental.pallas import tpu_sc as plsc

NC = 2
NS = 16
NW = NC * NS
L = 16

B = 16384
BPW = B // NW
CH = 128
NCH = BPW // CH

_mesh = plsc.VectorSubcoreMesh(core_axis_name="c", subcore_axis_name="s")


@functools.partial(
    pl.kernel,
    mesh=_mesh,
    compiler_params=pltpu.CompilerParams(
        allow_input_fusion=[True, False],
        disable_bounds_checks=True,
        disable_semaphore_checks=True,
    ),
    out_type=jax.ShapeDtypeStruct((B,), jnp.float32),
    scratch_types=[
        pltpu.VMEM((BPW,), jnp.int32),
        pltpu.VMEM((NCH, CH), jnp.int32),
        pltpu.VMEM((NCH, CH), jnp.float32),
        [pltpu.SemaphoreType.DMA] * NCH,
        [pltpu.SemaphoreType.DMA] * NCH,
        pltpu.SemaphoreType.DMA,
    ],
)
def _lookup(packed_hbm, table_hbm, out_hbm, p_v, idx_v, val_v, sem_in, sem_t, sem_o):
    wid = lax.axis_index("s") * NC + lax.axis_index("c")
    base = wid * BPW
    in_copies = [
        pltpu.async_copy(
            packed_hbm.at[pl.ds(base + j * CH, CH)],
            p_v.at[pl.ds(j * CH, CH)],
            sem_in[j],
        )
        for j in range(NCH)
    ]
    tbl_copies = []
    out_copies = []
    for j in range(NCH):
        in_copies[j].wait()
        row = idx_v.at[j]
        for i in range(CH // L):
            p = p_v[pl.ds(j * CH + i * L, L)]
            row[pl.ds(i * L, L)] = (p >> 16) * 1000 + (p & 0xFFFF)
        tbl_copies.append(
            pltpu.async_copy(table_hbm.at[row], val_v.at[j], sem_t[j])
        )
    for j in range(NCH):
        tbl_copies[j].wait()
        out_copies.append(
            pltpu.async_copy(
                val_v.at[j], out_hbm.at[pl.ds(base + j * CH, CH)], sem_o
            )
        )
    for c in out_copies:
        c.wait()


def kernel(xs, param_vec):
    packed = xs[:, 0] * 65536 + xs[:, 1]
    return _lookup(packed, param_vec)


# speedup vs baseline: 1.0013x; 1.0013x over previous
import functools

import jax
import jax.numpy as jnp
from jax import lax
from jax.experimental import pallas as pl
from jax.experimental.pallas import tpu as pltpu
from jax.experimental.pallas import tpu_sc as plsc

NC = 2
NS = 16
NW = NC * NS
L = 16

B = 16384
BPW = B // NW
CH = 128
NCH = BPW // CH

_mesh = plsc.VectorSubcoreMesh(core_axis_name="c", subcore_axis_name="s")


@functools.partial(
    pl.kernel,
    mesh=_mesh,
    compiler_params=pltpu.CompilerParams(allow_input_fusion=[True, False]),
    out_type=jax.ShapeDtypeStruct((B,), jnp.float32),
    scratch_types=[
        pltpu.VMEM((BPW,), jnp.int32),
        pltpu.VMEM((NCH, CH), jnp.int32),
        pltpu.VMEM((NCH, CH), jnp.float32),
        [pltpu.SemaphoreType.DMA] * NCH,
        [pltpu.SemaphoreType.DMA] * NCH,
        pltpu.SemaphoreType.DMA,
    ],
)
def _lookup(packed_hbm, table_hbm, out_hbm, p_v, idx_v, val_v, sem_in, sem_t, sem_o):
    wid = lax.axis_index("s") * NC + lax.axis_index("c")
    base = wid * BPW
    in_copies = [
        pltpu.async_copy(
            packed_hbm.at[pl.ds(base + j * CH, CH)],
            p_v.at[pl.ds(j * CH, CH)],
            sem_in[j],
        )
        for j in range(NCH)
    ]
    tbl_copies = []
    out_copies = []
    for j in range(NCH):
        in_copies[j].wait()
        row = idx_v.at[j]
        for i in range(CH // L):
            p = p_v[pl.ds(j * CH + i * L, L)]
            row[pl.ds(i * L, L)] = (p >> 16) * 1000 + (p & 0xFFFF)
        tbl_copies.append(
            pltpu.async_copy(table_hbm.at[row], val_v.at[j], sem_t[j])
        )
    for j in range(NCH):
        tbl_copies[j].wait()
        out_copies.append(
            pltpu.async_copy(
                val_v.at[j], out_hbm.at[pl.ds(base + j * CH, CH)], sem_o
            )
        )
    for c in out_copies:
        c.wait()


def kernel(xs, param_vec):
    packed = (xs[:, 0] << 16) | xs[:, 1]
    return _lookup(packed, param_vec)
